# baseline (speedup 1.0000x reference)
import functools

import jax
import jax.numpy as jnp
import numpy as np
from jax import lax
from jax.experimental import pallas as pl
from jax.experimental.pallas import tpu as pltpu
from jax.experimental.pallas import tpu_sc as plsc

N_NODES = 10000
N_EDGES = 320000
HID = 128
N_GRAPHS = 64

BLK = 2000
NBLK = N_NODES // BLK

NC, NS = 2, 16
NW = NC * NS
SUB = 64
CPT = 160
HEAVY = 0
CPT_N = 168
CPT_S = 2 * CPT - CPT_N
NBUF = 4
STG = 2
BCH = 8
WR = 624
VLANE = 16

_PERM = np.empty((HID,), dtype=np.int32)
for _j in range(HID // 32):
    for _k in range(16):
        _PERM[32 * _j + _k] = 32 * _j + 2 * _k
        _PERM[32 * _j + 16 + _k] = 32 * _j + 2 * _k + 1

_sc_mesh = plsc.VectorSubcoreMesh(core_axis_name="c", subcore_axis_name="s")


@functools.partial(
    pl.kernel,
    out_type=jax.ShapeDtypeStruct((NC, N_NODES, HID), jnp.float32),
    mesh=_sc_mesh,
    compiler_params=pltpu.CompilerParams(use_tc_tiling_on_sc=False),
    scratch_types=[
        pltpu.VMEM((3, BCH, SUB), jnp.int32),
        pltpu.VMEM((3, BCH, SUB), jnp.int32),
        pltpu.VMEM((3, BCH, SUB), jnp.float32),
        [pltpu.VMEM((SUB, HID // 2), jnp.int32) for _ in range(NBUF)],
        [pltpu.VMEM((SUB, HID), jnp.float32) for _ in range(STG)],
        [pltpu.SemaphoreType.DMA for _ in range(NBUF)],
        [pltpu.SemaphoreType.DMA for _ in range(STG)],
        pltpu.SemaphoreType.DMA,
        pltpu.VMEM_SHARED((N_NODES, HID), jnp.float32),
    ],
)
def _sc_agg(x_hbm, src_hbm, dst_hbm, mask_hbm, out_hbm,
            sbuf, dbuf, mbuf, rows, stag, gsem, ssem, isem, acc):
    cid = lax.axis_index("c")
    sid = lax.axis_index("s")
    rbase = sid * WR
    heavy = cid == HEAVY
    cpt = jnp.where(heavy, CPT_N, CPT_S)
    nbatch = cpt // BCH
    cbase = jnp.where(heavy, sid * CPT_N, NS * CPT_N + sid * CPT_S)

    def _load_batch(t, slot):
        pltpu.async_copy(src_hbm.at[pl.ds(cbase + t * BCH, BCH), :],
                         sbuf.at[slot], isem)
        pltpu.async_copy(dst_hbm.at[pl.ds(cbase + t * BCH, BCH), :],
                         dbuf.at[slot], isem)
        pltpu.async_copy(mask_hbm.at[pl.ds(cbase + t * BCH, BCH), :],
                         mbuf.at[slot], isem)

    def _wait_batch(t, slot):
        pltpu.make_async_copy(src_hbm.at[pl.ds(cbase + t * BCH, BCH), :],
                              sbuf.at[slot], isem).wait()
        pltpu.make_async_copy(dst_hbm.at[pl.ds(cbase + t * BCH, BCH), :],
                              dbuf.at[slot], isem).wait()
        pltpu.make_async_copy(mask_hbm.at[pl.ds(cbase + t * BCH, BCH), :],
                              mbuf.at[slot], isem).wait()

    def _drain_scatter(sg, slot, row):
        pltpu.make_async_copy(stag[sg], acc.at[dbuf.at[slot, row]],
                              ssem[sg]).wait()

    _load_batch(0, 0)
    _load_batch(1, 1)

    def _zrow(r, _):
        for j in range(HID // VLANE):
            stag[0][r, pl.ds(j * VLANE, VLANE)] = jnp.zeros((VLANE,),
                                                            jnp.float32)
        return 0
    lax.fori_loop(0, SUB, _zrow, 0)
    for q in range(WR // SUB):
        pltpu.sync_copy(stag[0], acc.at[pl.ds(rbase + q * SUB, SUB), :])
    pltpu.sync_copy(stag[0].at[pl.ds(0, WR - (WR // SUB) * SUB), :],
                    acc.at[pl.ds(rbase + (WR // SUB) * SUB,
                                 WR - (WR // SUB) * SUB), :])

    @pl.when(sid == NS - 1)
    def _zero_tail():
        pltpu.sync_copy(stag[0].at[pl.ds(0, N_NODES - NS * WR), :],
                        acc.at[pl.ds(NS * WR, N_NODES - NS * WR), :])
    plsc.subcore_barrier()

    def _scale(rv, st, slot, row):
        def _body(g, _):
            m16 = mbuf[slot, row, pl.ds(g * VLANE, VLANE)]
            for k in range(VLANE):
                mv = jnp.full((VLANE,), m16[k], jnp.float32)
                r = g * VLANE + k
                for j in range(HID // 32):
                    w = rv[r, pl.ds(VLANE * j, VLANE)]
                    a = lax.bitcast_convert_type(w << 16, jnp.float32)
                    b = lax.bitcast_convert_type(w & jnp.int32(-65536),
                                                 jnp.float32)
                    st[r, pl.ds(32 * j, VLANE)] = a * mv
                    st[r, pl.ds(32 * j + VLANE, VLANE)] = b * mv
            return 0
        lax.fori_loop(0, SUB // VLANE, _body, 0)

    _wait_batch(0, 0)
    _wait_batch(1, 1)
    for b in range(NBUF - 1):
        pltpu.async_copy(x_hbm.at[sbuf.at[0, b]], rows[b], gsem[b])

    def _batch_iter(t, _):
        slot_cur = lax.rem(t, 3)
        slot_nxt = lax.rem(t + 1, 3)
        slot_prv = lax.rem(t + 2, 3)

        @pl.when(jnp.logical_and(t >= 1, t + 1 < nbatch))
        def _wait_next():
            _wait_batch(t + 1, slot_nxt)

        @pl.when(t > 0)
        def _drain_prev():
            _drain_scatter(0, slot_prv, BCH - 2)
            _drain_scatter(1, slot_prv, BCH - 1)

        @pl.when(t + 2 < nbatch)
        def _issue_next():
            _load_batch(t + 2, slot_prv)

        for b in range(BCH):
            i = t * BCH + b
            bu = b % NBUF
            p = (bu + NBUF - 1) % NBUF
            sg = b % STG
            pltpu.make_async_copy(x_hbm.at[sbuf.at[slot_cur, b]], rows[bu],
                                  gsem[bu]).wait()
            if b >= STG:
                _drain_scatter(sg, slot_cur, b - STG)
            _scale(rows[bu], stag[sg], slot_cur, b)

            @pl.when(i + NBUF - 1 < cpt)
            def _prefetch():
                if b + NBUF - 1 < BCH:
                    idx_ref = sbuf.at[slot_cur, b + NBUF - 1]
                else:
                    idx_ref = sbuf.at[slot_nxt, b + NBUF - 1 - BCH]
                pltpu.async_copy(x_hbm.at[idx_ref], rows[p], gsem[p])

            pltpu.async_copy(stag[sg], acc.at[dbuf.at[slot_cur, b]],
                             ssem[sg], add=True)
        return 0
    lax.fori_loop(0, nbatch, _batch_iter, 0)
    _drain_scatter(0, lax.rem(nbatch - 1, 3), BCH - 2)
    _drain_scatter(1, lax.rem(nbatch - 1, 3), BCH - 1)
    plsc.subcore_barrier()
    pltpu.sync_copy(acc.at[pl.ds(rbase, WR), :],
                    out_hbm.at[cid, pl.ds(rbase, WR), :])

    @pl.when(sid == NS - 1)
    def _write_tail():
        pltpu.sync_copy(acc.at[pl.ds(NS * WR, N_NODES - NS * WR), :],
                        out_hbm.at[cid, pl.ds(NS * WR, N_NODES - NS * WR), :])


def _tc_body(batch_ref, part_ref, Wg_ref, bg_ref, W1_ref, b1_ref,
             W2_ref, b2_ref, out_ref, sums_ref, counts_ref):
    i = pl.program_id(0)

    @pl.when(i == 0)
    def _init():
        sums_ref[...] = jnp.zeros_like(sums_ref)
        counts_ref[...] = jnp.zeros_like(counts_ref)

    agg = part_ref[0] + part_ref[1]
    h = jnp.maximum(
        jnp.dot(agg, Wg_ref[...], preferred_element_type=jnp.float32)
        + bg_ref[...], 0.0)
    b = batch_ref[0]
    gids = lax.broadcasted_iota(jnp.int32, (N_GRAPHS, BLK), 0).astype(
        jnp.float32)
    onehot = (b == gids).astype(jnp.float32)
    sums_ref[...] += jnp.dot(onehot, h, preferred_element_type=jnp.float32)
    counts_ref[...] += jnp.sum(onehot, axis=1, keepdims=True)

    @pl.when(i == NBLK - 1)
    def _final():
        pool = sums_ref[...] / jnp.maximum(counts_ref[...], 1.0)
        t = jnp.dot(pool, W1_ref[...], preferred_element_type=jnp.float32) \
            + b1_ref[...]
        t = jnp.where(t > 0, t, jnp.exp(jnp.minimum(t, 0.0)) - 1.0)
        out_ref[...] = jnp.dot(t, W2_ref[...],
                               preferred_element_type=jnp.float32) + b2_ref[...]


@functools.partial(jax.jit)
def _tc_phase(part, batch_f, W_gnn, b_gnn, W1, b1, W2, b2):
    batch3 = batch_f.reshape(NBLK, 1, BLK)
    return pl.pallas_call(
        _tc_body,
        grid=(NBLK,),
        in_specs=[
            pl.BlockSpec((1, 1, BLK), lambda i: (i, 0, 0)),
            pl.BlockSpec((NC, BLK, HID), lambda i: (0, i, 0)),
            pl.BlockSpec((HID, HID), lambda i: (0, 0)),
            pl.BlockSpec((1, HID), lambda i: (0, 0)),
            pl.BlockSpec((HID, HID), lambda i: (0, 0)),
            pl.BlockSpec((1, HID), lambda i: (0, 0)),
            pl.BlockSpec((HID, 2), lambda i: (0, 0)),
            pl.BlockSpec((1, 2), lambda i: (0, 0)),
        ],
        out_specs=pl.BlockSpec((N_GRAPHS, 2), lambda i: (0, 0)),
        out_shape=jax.ShapeDtypeStruct((N_GRAPHS, 2), jnp.float32),
        scratch_shapes=[
            pltpu.VMEM((N_GRAPHS, HID), jnp.float32),
            pltpu.VMEM((N_GRAPHS, HID), jnp.float32),
        ],
    )(batch3, part, W_gnn, b_gnn.reshape(1, HID), W1,
      b1.reshape(1, HID), W2, b2.reshape(1, 2))


def kernel(x, edge_index, batch, mask, ids, W_gnn, b_gnn, W1, b1, W2, b2):
    pad = ((0, CPT * NW - N_EDGES // SUB), (0, 0))
    src = jnp.pad(edge_index[0].astype(jnp.int32).reshape(-1, SUB), pad)
    dst = jnp.pad(edge_index[1].astype(jnp.int32).reshape(-1, SUB), pad)
    maskp = jnp.pad(mask.reshape(-1, SUB), pad)
    u = lax.bitcast_convert_type(x, jnp.int32)
    b16 = (u + jnp.int32(0x7FFF) + ((u >> 16) & 1)) >> 16
    x32 = (b16[:, 0::2] & jnp.int32(0xFFFF)) | (b16[:, 1::2] << 16)
    part = _sc_agg(x32, src, dst, maskp)
    batch_f = batch.astype(jnp.float32)
    wg = W_gnn[jnp.asarray(_PERM)]
    out = _tc_phase(part, batch_f, wg, b_gnn, W1, b1, W2, b2)
    return jnp.squeeze(out)


# speedup vs baseline: 1.6643x; 1.6643x over previous
import functools

import jax
import jax.numpy as jnp
import numpy as np
from jax import lax
from jax.experimental import pallas as pl
from jax.experimental.pallas import tpu as pltpu
from jax.experimental.pallas import tpu_sc as plsc

N_NODES = 10000
N_EDGES = 320000
HID = 128
N_GRAPHS = 64

BLK = 2000
NBLK = N_NODES // BLK

NC, NS = 2, 16
NW = NC * NS
SUB = 64
CPT = 160
HEAVY = 0
CPT_N = 168
CPT_S = 2 * CPT - CPT_N
NBUF = 4
STG = 2
BCH = 8
WR = 624
VLANE = 16

_sc_mesh = plsc.VectorSubcoreMesh(core_axis_name="c", subcore_axis_name="s")


@functools.partial(
    pl.kernel,
    out_type=jax.ShapeDtypeStruct((NC, N_NODES, HID), jnp.float32),
    mesh=_sc_mesh,
    compiler_params=pltpu.CompilerParams(use_tc_tiling_on_sc=False),
    scratch_types=[
        pltpu.VMEM((3, BCH, SUB), jnp.int32),
        pltpu.VMEM((3, BCH, SUB), jnp.int32),
        pltpu.VMEM((3, BCH, SUB), jnp.float32),
        [pltpu.VMEM((SUB, HID // 2), jnp.int32) for _ in range(NBUF)],
        [pltpu.VMEM((SUB, HID), jnp.float32) for _ in range(STG)],
        [pltpu.SemaphoreType.DMA for _ in range(NBUF)],
        [pltpu.SemaphoreType.DMA for _ in range(STG)],
        pltpu.SemaphoreType.DMA,
        pltpu.VMEM_SHARED((N_NODES, HID), jnp.float32),
    ],
)
def _sc_agg(x_hbm, src_hbm, dst_hbm, mask_hbm, out_hbm,
            sbuf, dbuf, mbuf, rows, stag, gsem, ssem, isem, acc):
    cid = lax.axis_index("c")
    sid = lax.axis_index("s")
    rbase = sid * WR
    heavy = cid == HEAVY
    cpt = jnp.where(heavy, CPT_N, CPT_S)
    nbatch = cpt // BCH
    cbase = jnp.where(heavy, sid * CPT_N, NS * CPT_N + sid * CPT_S)

    def _load_batch(t, slot):
        pltpu.async_copy(src_hbm.at[pl.ds(cbase + t * BCH, BCH), :],
                         sbuf.at[slot], isem)
        pltpu.async_copy(dst_hbm.at[pl.ds(cbase + t * BCH, BCH), :],
                         dbuf.at[slot], isem)
        pltpu.async_copy(mask_hbm.at[pl.ds(cbase + t * BCH, BCH), :],
                         mbuf.at[slot], isem)

    def _wait_batch(t, slot):
        pltpu.make_async_copy(src_hbm.at[pl.ds(cbase + t * BCH, BCH), :],
                              sbuf.at[slot], isem).wait()
        pltpu.make_async_copy(dst_hbm.at[pl.ds(cbase + t * BCH, BCH), :],
                              dbuf.at[slot], isem).wait()
        pltpu.make_async_copy(mask_hbm.at[pl.ds(cbase + t * BCH, BCH), :],
                              mbuf.at[slot], isem).wait()

    def _drain_scatter(sg, slot, row):
        pltpu.make_async_copy(stag[sg], acc.at[dbuf.at[slot, row]],
                              ssem[sg]).wait()

    _load_batch(0, 0)
    _load_batch(1, 1)

    def _zrow(r, _):
        for j in range(HID // VLANE):
            stag[0][r, pl.ds(j * VLANE, VLANE)] = jnp.zeros((VLANE,),
                                                            jnp.float32)
        return 0
    lax.fori_loop(0, SUB, _zrow, 0)
    for q in range(WR // SUB):
        pltpu.sync_copy(stag[0], acc.at[pl.ds(rbase + q * SUB, SUB), :])
    pltpu.sync_copy(stag[0].at[pl.ds(0, WR - (WR // SUB) * SUB), :],
                    acc.at[pl.ds(rbase + (WR // SUB) * SUB,
                                 WR - (WR // SUB) * SUB), :])

    @pl.when(sid == NS - 1)
    def _zero_tail():
        pltpu.sync_copy(stag[0].at[pl.ds(0, N_NODES - NS * WR), :],
                        acc.at[pl.ds(NS * WR, N_NODES - NS * WR), :])
    plsc.subcore_barrier()

    def _scale(rv, st, slot, row):
        def _body(g, _):
            m16 = mbuf[slot, row, pl.ds(g * VLANE, VLANE)]
            for k in range(VLANE):
                mv = jnp.full((VLANE,), m16[k], jnp.float32)
                r = g * VLANE + k
                for j in range(HID // 32):
                    w = rv[r, pl.ds(VLANE * j, VLANE)]
                    a = lax.bitcast_convert_type(w << 16, jnp.float32)
                    b = lax.bitcast_convert_type(w & jnp.int32(-65536),
                                                 jnp.float32)
                    st[r, pl.ds(VLANE * j, VLANE)] = a * mv
                    st[r, pl.ds(HID // 2 + VLANE * j, VLANE)] = b * mv
            return 0
        lax.fori_loop(0, SUB // VLANE, _body, 0)

    _wait_batch(0, 0)
    _wait_batch(1, 1)
    for b in range(NBUF - 1):
        pltpu.async_copy(x_hbm.at[sbuf.at[0, b]], rows[b], gsem[b])

    def _batch_iter(t, _):
        slot_cur = lax.rem(t, 3)
        slot_nxt = lax.rem(t + 1, 3)
        slot_prv = lax.rem(t + 2, 3)

        @pl.when(jnp.logical_and(t >= 1, t + 1 < nbatch))
        def _wait_next():
            _wait_batch(t + 1, slot_nxt)

        @pl.when(t > 0)
        def _drain_prev():
            _drain_scatter(0, slot_prv, BCH - 2)
            _drain_scatter(1, slot_prv, BCH - 1)

        @pl.when(t + 2 < nbatch)
        def _issue_next():
            _load_batch(t + 2, slot_prv)

        for b in range(BCH):
            i = t * BCH + b
            bu = b % NBUF
            p = (bu + NBUF - 1) % NBUF
            sg = b % STG
            pltpu.make_async_copy(x_hbm.at[sbuf.at[slot_cur, b]], rows[bu],
                                  gsem[bu]).wait()
            if b >= STG:
                _drain_scatter(sg, slot_cur, b - STG)
            _scale(rows[bu], stag[sg], slot_cur, b)

            @pl.when(i + NBUF - 1 < cpt)
            def _prefetch():
                if b + NBUF - 1 < BCH:
                    idx_ref = sbuf.at[slot_cur, b + NBUF - 1]
                else:
                    idx_ref = sbuf.at[slot_nxt, b + NBUF - 1 - BCH]
                pltpu.async_copy(x_hbm.at[idx_ref], rows[p], gsem[p])

            pltpu.async_copy(stag[sg], acc.at[dbuf.at[slot_cur, b]],
                             ssem[sg], add=True)
        return 0
    lax.fori_loop(0, nbatch, _batch_iter, 0)
    _drain_scatter(0, lax.rem(nbatch - 1, 3), BCH - 2)
    _drain_scatter(1, lax.rem(nbatch - 1, 3), BCH - 1)
    plsc.subcore_barrier()
    pltpu.sync_copy(acc.at[pl.ds(rbase, WR), :],
                    out_hbm.at[cid, pl.ds(rbase, WR), :])

    @pl.when(sid == NS - 1)
    def _write_tail():
        pltpu.sync_copy(acc.at[pl.ds(NS * WR, N_NODES - NS * WR), :],
                        out_hbm.at[cid, pl.ds(NS * WR, N_NODES - NS * WR), :])


def _tc_body(batch_ref, part_ref, Wg_ref, bg_ref, W1_ref, b1_ref,
             W2_ref, b2_ref, out_ref, sums_ref, counts_ref):
    i = pl.program_id(0)

    @pl.when(i == 0)
    def _init():
        sums_ref[...] = jnp.zeros_like(sums_ref)
        counts_ref[...] = jnp.zeros_like(counts_ref)

    agg = part_ref[0] + part_ref[1]
    h = jnp.maximum(
        jnp.dot(agg, Wg_ref[...], preferred_element_type=jnp.float32)
        + bg_ref[...], 0.0)
    b = batch_ref[0]
    gids = lax.broadcasted_iota(jnp.int32, (N_GRAPHS, BLK), 0).astype(
        jnp.float32)
    onehot = (b == gids).astype(jnp.float32)
    sums_ref[...] += jnp.dot(onehot, h, preferred_element_type=jnp.float32)
    counts_ref[...] += jnp.sum(onehot, axis=1, keepdims=True)

    @pl.when(i == NBLK - 1)
    def _final():
        pool = sums_ref[...] / jnp.maximum(counts_ref[...], 1.0)
        t = jnp.dot(pool, W1_ref[...], preferred_element_type=jnp.float32) \
            + b1_ref[...]
        t = jnp.where(t > 0, t, jnp.exp(jnp.minimum(t, 0.0)) - 1.0)
        out_ref[...] = jnp.dot(t, W2_ref[...],
                               preferred_element_type=jnp.float32) + b2_ref[...]


@functools.partial(jax.jit)
def _tc_phase(part, batch_f, W_gnn, b_gnn, W1, b1, W2, b2):
    batch3 = batch_f.reshape(NBLK, 1, BLK)
    return pl.pallas_call(
        _tc_body,
        grid=(NBLK,),
        in_specs=[
            pl.BlockSpec((1, 1, BLK), lambda i: (i, 0, 0)),
            pl.BlockSpec((NC, BLK, HID), lambda i: (0, i, 0)),
            pl.BlockSpec((HID, HID), lambda i: (0, 0)),
            pl.BlockSpec((1, HID), lambda i: (0, 0)),
            pl.BlockSpec((HID, HID), lambda i: (0, 0)),
            pl.BlockSpec((1, HID), lambda i: (0, 0)),
            pl.BlockSpec((HID, 2), lambda i: (0, 0)),
            pl.BlockSpec((1, 2), lambda i: (0, 0)),
        ],
        out_specs=pl.BlockSpec((N_GRAPHS, 2), lambda i: (0, 0)),
        out_shape=jax.ShapeDtypeStruct((N_GRAPHS, 2), jnp.float32),
        scratch_shapes=[
            pltpu.VMEM((N_GRAPHS, HID), jnp.float32),
            pltpu.VMEM((N_GRAPHS, HID), jnp.float32),
        ],
    )(batch3, part, W_gnn, b_gnn.reshape(1, HID), W1,
      b1.reshape(1, HID), W2, b2.reshape(1, 2))


def kernel(x, edge_index, batch, mask, ids, W_gnn, b_gnn, W1, b1, W2, b2):
    pad = ((0, CPT * NW - N_EDGES // SUB), (0, 0))
    src = jnp.pad(edge_index[0].astype(jnp.int32).reshape(-1, SUB), pad)
    dst = jnp.pad(edge_index[1].astype(jnp.int32).reshape(-1, SUB), pad)
    maskp = jnp.pad(mask.reshape(-1, SUB), pad)
    u = lax.bitcast_convert_type(x, jnp.int32)
    b16 = (u + jnp.int32(0x7FFF) + ((u >> 16) & 1)) >> 16
    x32 = (b16[:, :HID // 2] & jnp.int32(0xFFFF)) | (b16[:, HID // 2:] << 16)
    part = _sc_agg(x32, src, dst, maskp)
    batch_f = batch.astype(jnp.float32)
    out = _tc_phase(part, batch_f, W_gnn, b_gnn, W1, b1, W2, b2)
    return jnp.squeeze(out)


# speedup vs baseline: 1.7554x; 1.0547x over previous
import functools

import jax
import jax.numpy as jnp
import numpy as np
from jax import lax
from jax.experimental import pallas as pl
from jax.experimental.pallas import tpu as pltpu
from jax.experimental.pallas import tpu_sc as plsc

N_NODES = 10000
N_EDGES = 320000
HID = 128
N_GRAPHS = 64

BLK = 2000
NBLK = N_NODES // BLK

NC, NS = 2, 16
NW = NC * NS
SUB = 64
CPT = 160
HEAVY = 0
CPT_N = 168
CPT_S = 2 * CPT - CPT_N
NBUF = 4
STG = 2
BCH = 8
WR = 624
VLANE = 16

_sc_mesh = plsc.VectorSubcoreMesh(core_axis_name="c", subcore_axis_name="s")


@functools.partial(
    pl.kernel,
    out_type=jax.ShapeDtypeStruct((NC, N_NODES, HID), jnp.float32),
    mesh=_sc_mesh,
    compiler_params=pltpu.CompilerParams(use_tc_tiling_on_sc=False),
    scratch_types=[
        pltpu.VMEM((3, BCH, SUB), jnp.int32),
        pltpu.VMEM((3, BCH, SUB), jnp.int32),
        pltpu.VMEM((3, BCH, SUB), jnp.float32),
        [pltpu.VMEM((SUB, HID // 2), jnp.int32) for _ in range(NBUF)],
        [pltpu.VMEM((SUB, HID), jnp.float32) for _ in range(STG)],
        [pltpu.SemaphoreType.DMA for _ in range(NBUF)],
        [pltpu.SemaphoreType.DMA for _ in range(STG)],
        pltpu.SemaphoreType.DMA,
        pltpu.VMEM_SHARED((N_NODES, HID), jnp.float32),
    ],
)
def _sc_agg(x_hbm, ei_hbm, mask_hbm, out_hbm,
            sbuf, dbuf, mbuf, rows, stag, gsem, ssem, isem, acc):
    cid = lax.axis_index("c")
    sid = lax.axis_index("s")
    rbase = sid * WR
    heavy = cid == HEAVY
    cpt = jnp.where(heavy, CPT_N, CPT_S)
    nbatch = cpt // BCH
    cbase = jnp.where(heavy, sid * CPT_N, NS * CPT_N + sid * CPT_S)

    def _load_batch(t, slot):
        pltpu.async_copy(ei_hbm.at[0, pl.ds(cbase + t * BCH, BCH), :],
                         sbuf.at[slot], isem)
        pltpu.async_copy(ei_hbm.at[1, pl.ds(cbase + t * BCH, BCH), :],
                         dbuf.at[slot], isem)
        pltpu.async_copy(mask_hbm.at[pl.ds(cbase + t * BCH, BCH), :],
                         mbuf.at[slot], isem)

    def _wait_batch(t, slot):
        pltpu.make_async_copy(ei_hbm.at[0, pl.ds(cbase + t * BCH, BCH), :],
                              sbuf.at[slot], isem).wait()
        pltpu.make_async_copy(ei_hbm.at[1, pl.ds(cbase + t * BCH, BCH), :],
                              dbuf.at[slot], isem).wait()
        pltpu.make_async_copy(mask_hbm.at[pl.ds(cbase + t * BCH, BCH), :],
                              mbuf.at[slot], isem).wait()

    def _drain_scatter(sg, slot, row):
        pltpu.make_async_copy(stag[sg], acc.at[dbuf.at[slot, row]],
                              ssem[sg]).wait()

    _load_batch(0, 0)
    _load_batch(1, 1)

    def _zrow(r, _):
        for j in range(HID // VLANE):
            stag[0][r, pl.ds(j * VLANE, VLANE)] = jnp.zeros((VLANE,),
                                                            jnp.float32)
        return 0
    lax.fori_loop(0, SUB, _zrow, 0)
    for q in range(WR // SUB):
        pltpu.sync_copy(stag[0], acc.at[pl.ds(rbase + q * SUB, SUB), :])
    pltpu.sync_copy(stag[0].at[pl.ds(0, WR - (WR // SUB) * SUB), :],
                    acc.at[pl.ds(rbase + (WR // SUB) * SUB,
                                 WR - (WR // SUB) * SUB), :])

    @pl.when(sid == NS - 1)
    def _zero_tail():
        pltpu.sync_copy(stag[0].at[pl.ds(0, N_NODES - NS * WR), :],
                        acc.at[pl.ds(NS * WR, N_NODES - NS * WR), :])
    plsc.subcore_barrier()

    def _scale(rv, st, slot, row):
        def _body(g, _):
            m16 = mbuf[slot, row, pl.ds(g * VLANE, VLANE)]
            for k in range(VLANE):
                mv = jnp.full((VLANE,), m16[k], jnp.float32)
                r = g * VLANE + k
                for j in range(HID // 32):
                    w = rv[r, pl.ds(VLANE * j, VLANE)]
                    a = lax.bitcast_convert_type(w << 16, jnp.float32)
                    b = lax.bitcast_convert_type(w & jnp.int32(-65536),
                                                 jnp.float32)
                    st[r, pl.ds(VLANE * j, VLANE)] = a * mv
                    st[r, pl.ds(HID // 2 + VLANE * j, VLANE)] = b * mv
            return 0
        lax.fori_loop(0, SUB // VLANE, _body, 0)

    _wait_batch(0, 0)
    _wait_batch(1, 1)
    for b in range(NBUF - 1):
        pltpu.async_copy(x_hbm.at[sbuf.at[0, b]], rows[b], gsem[b])

    def _batch_iter(t, _):
        slot_cur = lax.rem(t, 3)
        slot_nxt = lax.rem(t + 1, 3)
        slot_prv = lax.rem(t + 2, 3)

        @pl.when(jnp.logical_and(t >= 1, t + 1 < nbatch))
        def _wait_next():
            _wait_batch(t + 1, slot_nxt)

        @pl.when(t > 0)
        def _drain_prev():
            _drain_scatter(0, slot_prv, BCH - 2)
            _drain_scatter(1, slot_prv, BCH - 1)

        @pl.when(t + 2 < nbatch)
        def _issue_next():
            _load_batch(t + 2, slot_prv)

        for b in range(BCH):
            i = t * BCH + b
            bu = b % NBUF
            p = (bu + NBUF - 1) % NBUF
            sg = b % STG
            pltpu.make_async_copy(x_hbm.at[sbuf.at[slot_cur, b]], rows[bu],
                                  gsem[bu]).wait()
            if b >= STG:
                _drain_scatter(sg, slot_cur, b - STG)
            _scale(rows[bu], stag[sg], slot_cur, b)

            @pl.when(i + NBUF - 1 < cpt)
            def _prefetch():
                if b + NBUF - 1 < BCH:
                    idx_ref = sbuf.at[slot_cur, b + NBUF - 1]
                else:
                    idx_ref = sbuf.at[slot_nxt, b + NBUF - 1 - BCH]
                pltpu.async_copy(x_hbm.at[idx_ref], rows[p], gsem[p])

            pltpu.async_copy(stag[sg], acc.at[dbuf.at[slot_cur, b]],
                             ssem[sg], add=True)
        return 0
    lax.fori_loop(0, nbatch, _batch_iter, 0)
    _drain_scatter(0, lax.rem(nbatch - 1, 3), BCH - 2)
    _drain_scatter(1, lax.rem(nbatch - 1, 3), BCH - 1)
    plsc.subcore_barrier()
    pltpu.sync_copy(acc.at[pl.ds(rbase, WR), :],
                    out_hbm.at[cid, pl.ds(rbase, WR), :])

    @pl.when(sid == NS - 1)
    def _write_tail():
        pltpu.sync_copy(acc.at[pl.ds(NS * WR, N_NODES - NS * WR), :],
                        out_hbm.at[cid, pl.ds(NS * WR, N_NODES - NS * WR), :])


def _tc_body(batch_ref, part_ref, Wg_ref, bg_ref, W1_ref, b1_ref,
             W2_ref, b2_ref, out_ref, sums_ref, counts_ref):
    i = pl.program_id(0)

    @pl.when(i == 0)
    def _init():
        sums_ref[...] = jnp.zeros_like(sums_ref)
        counts_ref[...] = jnp.zeros_like(counts_ref)

    agg = part_ref[0] + part_ref[1]
    h = jnp.maximum(
        jnp.dot(agg, Wg_ref[...], preferred_element_type=jnp.float32)
        + bg_ref[...], 0.0)
    b = batch_ref[0]
    gids = lax.broadcasted_iota(jnp.int32, (N_GRAPHS, BLK), 0).astype(
        jnp.float32)
    onehot = (b == gids).astype(jnp.float32)
    sums_ref[...] += jnp.dot(onehot, h, preferred_element_type=jnp.float32)
    counts_ref[...] += jnp.sum(onehot, axis=1, keepdims=True)

    @pl.when(i == NBLK - 1)
    def _final():
        pool = sums_ref[...] / jnp.maximum(counts_ref[...], 1.0)
        t = jnp.dot(pool, W1_ref[...], preferred_element_type=jnp.float32) \
            + b1_ref[...]
        t = jnp.where(t > 0, t, jnp.exp(jnp.minimum(t, 0.0)) - 1.0)
        out_ref[...] = jnp.dot(t, W2_ref[...],
                               preferred_element_type=jnp.float32) + b2_ref[...]


@functools.partial(jax.jit)
def _tc_phase(part, batch_f, W_gnn, b_gnn, W1, b1, W2, b2):
    batch3 = batch_f.reshape(NBLK, 1, BLK)
    return pl.pallas_call(
        _tc_body,
        grid=(NBLK,),
        in_specs=[
            pl.BlockSpec((1, 1, BLK), lambda i: (i, 0, 0)),
            pl.BlockSpec((NC, BLK, HID), lambda i: (0, i, 0)),
            pl.BlockSpec((HID, HID), lambda i: (0, 0)),
            pl.BlockSpec((1, HID), lambda i: (0, 0)),
            pl.BlockSpec((HID, HID), lambda i: (0, 0)),
            pl.BlockSpec((1, HID), lambda i: (0, 0)),
            pl.BlockSpec((HID, 2), lambda i: (0, 0)),
            pl.BlockSpec((1, 2), lambda i: (0, 0)),
        ],
        out_specs=pl.BlockSpec((N_GRAPHS, 2), lambda i: (0, 0)),
        out_shape=jax.ShapeDtypeStruct((N_GRAPHS, 2), jnp.float32),
        scratch_shapes=[
            pltpu.VMEM((N_GRAPHS, HID), jnp.float32),
            pltpu.VMEM((N_GRAPHS, HID), jnp.float32),
        ],
    )(batch3, part, W_gnn, b_gnn.reshape(1, HID), W1,
      b1.reshape(1, HID), W2, b2.reshape(1, 2))


def kernel(x, edge_index, batch, mask, ids, W_gnn, b_gnn, W1, b1, W2, b2):
    padrows = CPT * NW - N_EDGES // SUB
    ei = jnp.pad(edge_index.astype(jnp.int32).reshape(2, -1, SUB),
                 ((0, 0), (0, padrows), (0, 0)))
    maskp = jnp.pad(mask.reshape(-1, SUB), ((0, padrows), (0, 0)))
    u = lax.bitcast_convert_type(x, jnp.int32)
    b16 = (u + jnp.int32(0x7FFF) + ((u >> 16) & 1)) >> 16
    x32 = (b16[:, :HID // 2] & jnp.int32(0xFFFF)) | (b16[:, HID // 2:] << 16)
    part = _sc_agg(x32, ei, maskp)
    batch_f = batch.astype(jnp.float32)
    out = _tc_phase(part, batch_f, W_gnn, b_gnn, W1, b1, W2, b2)
    return jnp.squeeze(out)
